# baseline (speedup 1.0000x reference)
import jax
import jax.numpy as jnp
from jax import lax
from jax.experimental import pallas as pl
from jax.experimental.pallas import tpu as pltpu
from jax.experimental.pallas import tpu_sc as plsc

f32 = jnp.float32
i32 = jnp.int32

NU = 50000
NI = 50000
NN = NU + NI
D = 32
H = 16
NLAYERS = 3
E = 1600000
B = 4096

NC = 2
NS = 16

SUB = 128
NSTREAM = 16
C = SUB * NSTREAM
CHUNKS = 50
EPAD = NS * CHUNKS * C

NH = NN // 2
ACC_ROWS = 50016
ZPT = 3128
ZPT_LAST = ACC_ROWS - (NS - 1) * ZPT
WPT = 3128
WPT_LAST = NH - (NS - 1) * WPT

_mesh = plsc.VectorSubcoreMesh(
    core_axis_name="c", subcore_axis_name="s", num_cores=NC, num_subcores=NS
)

_BCAST_DN = lax.GatherDimensionNumbers(
    offset_dims=(), collapsed_slice_dims=(0,), start_index_map=(0,))


def _bcast_lane(v, lane):
    idx = jnp.full((16, 1), lane, i32)
    return lax.gather(v, idx, _BCAST_DN, (1,),
                      mode=lax.GatherScatterMode.PROMISE_IN_BOUNDS)


def _layer_body(h0, h1, eidx, eval_, o0, o1,
                cv0, cv1, dv0, dv1, vv0, vv1, rv0, rv1, acc,
                sg0, sg1, ss0, ss1):
    c = lax.axis_index("c")
    s = lax.axis_index("s")
    cv = [cv0, cv1]
    dv = [dv0, dv1]
    vv = [vv0, vv1]
    rv = [rv0, rv1]
    sg = [sg0, sg1]
    ss = [ss0, ss1]

    def _transform(bi, p):
        lo = p * NH

        @pl.loop(0, C, step=16)
        def _t(j0):
            rvec = dv[bi][pl.ds(j0, 16)]
            inr = (rvec >= lo) & (rvec < lo + NH)
            dv[bi][pl.ds(j0, 16)] = jnp.where(inr, rvec - lo, NH)

    def _stage(bi, ch, p):
        pltpu.sync_copy(eidx.at[s, ch, 0], cv[bi])
        pltpu.sync_copy(eidx.at[s, ch, 1], dv[bi])
        pltpu.sync_copy(eval_.at[s, ch], vv[bi])
        _transform(bi, p)

    def _fire_gather(bi):
        @pl.when(c == 0)
        def _g0():
            pltpu.async_copy(h0.at[cv[bi]], rv[bi], sg[bi])

        @pl.when(c == 1)
        def _g1():
            pltpu.async_copy(h1.at[cv[bi]], rv[bi], sg[bi])

    def _wait_gather(bi):
        pltpu.make_async_copy(h0.at[cv[bi]], rv[bi], sg[bi]).wait()

    def _fire_scatter(bi):
        pltpu.async_copy(rv[bi], acc.at[dv[bi]], ss[bi], add=True)

    def _wait_scatter(bi):
        pltpu.make_async_copy(rv[bi], acc.at[dv[bi]], ss[bi]).wait()

    lane = lax.iota(i32, 16)

    @pl.loop(0, 2)
    def _pass(p):
        @pl.loop(0, C)
        def _zf(j):
            rv1[j, :] = jnp.zeros((16,), f32)

        zb = s * ZPT
        pltpu.sync_copy(rv1, acc.at[pl.ds(zb, C)])

        @pl.when(s < NS - 1)
        def _zr1():
            pltpu.sync_copy(rv1.at[pl.ds(0, ZPT - C)],
                            acc.at[pl.ds(zb + C, ZPT - C)])

        @pl.when(s == NS - 1)
        def _zr2():
            pltpu.sync_copy(rv1.at[pl.ds(0, ZPT_LAST - C)],
                            acc.at[pl.ds(zb + C, ZPT_LAST - C)])

        _stage(0, 0, p)
        _fire_gather(0)

        plsc.subcore_barrier()

        @pl.loop(0, CHUNKS, step=2)
        def _chunk2(ch0):
            for b in range(2):
                ch = ch0 + b
                nb = 1 - b

                @pl.when(ch + 1 < CHUNKS)
                def _fire_next():
                    @pl.when(ch >= 1)
                    def _ws():
                        _wait_scatter(nb)

                    _stage(nb, ch + 1, p)
                    _fire_gather(nb)

                _wait_gather(b)

                @pl.loop(0, C, step=16)
                def _scale(jj):
                    v = vv[b][pl.ds(jj, 16)]
                    eidx16 = jj + lane
                    for k2 in range(H):
                        kf = jnp.full((16,), k2, i32)
                        g = plsc.load_gather(rv[b], [eidx16, kf])
                        plsc.store_scatter(rv[b], [eidx16, kf], g * v)

                _fire_scatter(b)

        _wait_scatter(0)
        _wait_scatter(1)
        plsc.subcore_barrier()

        def _writeback(dst):
            base = s * WPT
            pltpu.sync_copy(acc.at[pl.ds(base, C)],
                            dst.at[pl.ds(p * NH + base, C)])

            @pl.when(s < NS - 1)
            def _r1():
                pltpu.sync_copy(acc.at[pl.ds(base + C, WPT - C)],
                                dst.at[pl.ds(p * NH + base + C, WPT - C)])

            @pl.when(s == NS - 1)
            def _r2():
                pltpu.sync_copy(acc.at[pl.ds(base + C, WPT_LAST - C)],
                                dst.at[pl.ds(p * NH + base + C, WPT_LAST - C)])

        @pl.when(c == 0)
        def _w0():
            _writeback(o0)

        @pl.when(c == 1)
        def _w1():
            _writeback(o1)


_layer = pl.kernel(
    _layer_body,
    out_type=(
        jax.ShapeDtypeStruct((NN, H), f32),
        jax.ShapeDtypeStruct((NN, H), f32),
    ),
    mesh=_mesh,
    compiler_params=pltpu.CompilerParams(use_tc_tiling_on_sc=False, needs_layout_passes=False),
    scratch_types=[
        pltpu.VMEM((C,), i32),
        pltpu.VMEM((C,), i32),
        pltpu.VMEM((C,), i32),
        pltpu.VMEM((C,), i32),
        pltpu.VMEM((C,), f32),
        pltpu.VMEM((C,), f32),
        pltpu.VMEM((C, H), f32),
        pltpu.VMEM((C, H), f32),
        pltpu.VMEM_SHARED((ACC_ROWS, H), f32),
        pltpu.SemaphoreType.DMA,
        pltpu.SemaphoreType.DMA,
        pltpu.SemaphoreType.DMA,
        pltpu.SemaphoreType.DMA,
    ],
)

BPW = B // (NC * NS)


def _score_body(e00, e01, e10, e11, e20, e21, e30, e31, uids, iids, out,
                idx_u, idx_i, gbuf, sc_v, sem):
    c = lax.axis_index("c")
    s = lax.axis_index("s")
    w = s * NC + c

    pltpu.sync_copy(uids.at[pl.ds(w * BPW, BPW)], idx_u)
    pltpu.sync_copy(iids.at[pl.ds(w * BPW, BPW)], idx_i)

    tabs = [e00, e01, e10, e11, e20, e21, e30, e31]
    descs = []
    for t in range(8):
        descs.append(pltpu.async_copy(tabs[t].at[idx_u], gbuf.at[t], sem))
        descs.append(pltpu.async_copy(tabs[t].at[idx_i], gbuf.at[8 + t], sem))
    for d in descs:
        d.wait()

    @pl.loop(0, BPW, step=16)
    def _dot(j0):
        lane = lax.iota(i32, 16)
        tot = jnp.zeros((16,), f32)
        for jj in range(16):
            j = j0 + jj
            u0 = gbuf[0, j, :] + gbuf[2, j, :] + gbuf[4, j, :] + gbuf[6, j, :]
            u1 = gbuf[1, j, :] + gbuf[3, j, :] + gbuf[5, j, :] + gbuf[7, j, :]
            i0 = gbuf[8, j, :] + gbuf[10, j, :] + gbuf[12, j, :] + gbuf[14, j, :]
            i1 = gbuf[9, j, :] + gbuf[11, j, :] + gbuf[13, j, :] + gbuf[15, j, :]
            p = u0 * i0 + u1 * i1
            ssum = jnp.sum(p) * (1.0 / 16.0)
            tot = jnp.where(lane == jj, lax.broadcast_in_dim(ssum, (16,), ()), tot)
        sc_v[pl.ds(j0, 16)] = tot

    pltpu.sync_copy(sc_v, out.at[pl.ds(w * BPW, BPW)])


_score = pl.kernel(
    _score_body,
    out_type=jax.ShapeDtypeStruct((B,), f32),
    mesh=_mesh,
    compiler_params=pltpu.CompilerParams(use_tc_tiling_on_sc=False, needs_layout_passes=False),
    scratch_types=[
        pltpu.VMEM((BPW,), i32),
        pltpu.VMEM((BPW,), i32),
        pltpu.VMEM((16, BPW, H), f32),
        pltpu.VMEM((BPW,), f32),
        pltpu.SemaphoreType.DMA,
    ],
)


def kernel(user_ids, item_ids, adj_indices, adj_values, user_emb_w, item_emb_w):
    row = adj_indices[0].astype(i32)
    col = adj_indices[1].astype(i32)
    pad = EPAD - E
    colp = jnp.concatenate([col, jnp.zeros((pad,), i32)])
    rowp = jnp.concatenate([row, jnp.zeros((pad,), i32)])
    valp = jnp.concatenate([adj_values.astype(f32), jnp.zeros((pad,), f32)])
    col2 = colp.reshape(NS, CHUNKS, C)
    row2 = rowp.reshape(NS, CHUNKS, C)
    eidx = jnp.stack([col2, row2], axis=2)
    eval_ = valp.reshape(NS, CHUNKS, C)

    a0 = jnp.concatenate([user_emb_w[:, :H], item_emb_w[:, :H]], axis=0)
    a1 = jnp.concatenate([user_emb_w[:, H:], item_emb_w[:, H:]], axis=0)
    embs = [(a0, a1)]
    for _ in range(NLAYERS):
        a0, a1 = _layer(a0, a1, eidx, eval_)
        embs.append((a0, a1))

    uids = user_ids.astype(i32)
    iids = item_ids.astype(i32) + NU
    return _score(embs[0][0], embs[0][1], embs[1][0], embs[1][1],
                  embs[2][0], embs[2][1], embs[3][0], embs[3][1],
                  uids, iids)


# speedup vs baseline: 1.2752x; 1.2752x over previous
import jax
import jax.numpy as jnp
from jax import lax
from jax.experimental import pallas as pl
from jax.experimental.pallas import tpu as pltpu
from jax.experimental.pallas import tpu_sc as plsc

f32 = jnp.float32
i32 = jnp.int32

NU = 50000
NI = 50000
NN = NU + NI
H = 16
NLAYERS = 3
E = 1600000
B = 4096

NC = 2
NS = 16
NW = NC * NS

SUB = 128
NSTREAM = 16
C = SUB * NSTREAM
PCH = 25
EPAD = NW * PCH * C
REG = 57344

NH = NN // 2
ACC_ROWS = NH
ZPT = 3128
ZPT_LAST = NH - (NS - 1) * ZPT

_mesh = plsc.VectorSubcoreMesh(
    core_axis_name="c", subcore_axis_name="s", num_cores=NC, num_subcores=NS
)
_params = pltpu.CompilerParams(use_tc_tiling_on_sc=False,
                               needs_layout_passes=False)


def _partition_body(pe_idx, pe_val, bcol, brow, bval, bcnt,
                    icol, irow, ival, oc0, or0, ov0, oc1, or1, ov1, cntv):
    c = lax.axis_index("c")
    s = lax.axis_index("s")
    w = s * NC + c
    lane = lax.iota(i32, 16)
    zi = jnp.zeros((16,), i32)
    zf = jnp.zeros((16,), f32)

    @pl.loop(0, PCH, init_carry=(jnp.array(0, i32), jnp.array(0, i32)))
    def _chunks(ch, g):
        g0, g1 = g
        pltpu.sync_copy(pe_idx.at[w, ch, 0], icol)
        pltpu.sync_copy(pe_idx.at[w, ch, 1], irow)
        pltpu.sync_copy(pe_val.at[w, ch], ival)

        @pl.loop(0, C, step=16,
                 init_carry=(jnp.array(0, i32), jnp.array(0, i32)))
        def _groups(j0, pp):
            p0, p1 = pp
            cvec = icol[pl.ds(j0, 16)]
            rvec = irow[pl.ds(j0, 16)]
            vvec = ival[pl.ds(j0, 16)]
            m = rvec < NH
            cum = plsc.cumsum(jnp.where(m, 1, 0))
            pos0 = cum + (p0 - 1)
            plsc.store_scatter(oc0, [pos0], cvec, mask=m)
            plsc.store_scatter(or0, [pos0], rvec, mask=m)
            plsc.store_scatter(ov0, [pos0], vvec, mask=m)
            m1 = jnp.logical_not(m)
            pos1 = (lane + 1 - cum) + (p1 - 1)
            plsc.store_scatter(oc1, [pos1], cvec, mask=m1)
            plsc.store_scatter(or1, [pos1], rvec - NH, mask=m1)
            plsc.store_scatter(ov1, [pos1], vvec, mask=m1)
            n0 = jnp.max(cum)
            return (p0 + n0, p1 + (16 - n0))

        p0, p1 = _groups
        g0 = pl.multiple_of(g0, 128)
        g1 = pl.multiple_of(g1, 128)
        pp0 = (p0 + 127) & (-128)
        pp1 = (p1 + 127) & (-128)
        for g8 in range(8):
            posv = p0 + g8 * 16 + lane
            mk = posv < pp0
            plsc.store_scatter(oc0, [posv], zi, mask=mk)
            plsc.store_scatter(or0, [posv], zi, mask=mk)
            plsc.store_scatter(ov0, [posv], zf, mask=mk)
            posw = p1 + g8 * 16 + lane
            mk1 = posw < pp1
            plsc.store_scatter(oc1, [posw], zi, mask=mk1)
            plsc.store_scatter(or1, [posw], zi, mask=mk1)
            plsc.store_scatter(ov1, [posw], zf, mask=mk1)
        pltpu.sync_copy(oc0, bcol.at[0, w, pl.ds(g0, C)])
        pltpu.sync_copy(or0, brow.at[0, w, pl.ds(g0, C)])
        pltpu.sync_copy(ov0, bval.at[0, w, pl.ds(g0, C)])
        pltpu.sync_copy(oc1, bcol.at[1, w, pl.ds(g1, C)])
        pltpu.sync_copy(or1, brow.at[1, w, pl.ds(g1, C)])
        pltpu.sync_copy(ov1, bval.at[1, w, pl.ds(g1, C)])
        return (g0 + pp0, g1 + pp1)

    g0, g1 = _chunks
    g0 = pl.multiple_of(g0, 128)
    g1 = pl.multiple_of(g1, 128)

    @pl.loop(0, C, step=16)
    def _zz(j0):
        oc0[pl.ds(j0, 16)] = zi
        or0[pl.ds(j0, 16)] = zi
        ov0[pl.ds(j0, 16)] = zf

    pltpu.sync_copy(oc0, bcol.at[0, w, pl.ds(g0, C)])
    pltpu.sync_copy(or0, brow.at[0, w, pl.ds(g0, C)])
    pltpu.sync_copy(ov0, bval.at[0, w, pl.ds(g0, C)])
    pltpu.sync_copy(oc0, bcol.at[1, w, pl.ds(g1, C)])
    pltpu.sync_copy(or0, brow.at[1, w, pl.ds(g1, C)])
    pltpu.sync_copy(ov0, bval.at[1, w, pl.ds(g1, C)])

    cntv[pl.ds(0, 16)] = lax.broadcast_in_dim(g0, (16,), ())
    pltpu.sync_copy(cntv, bcnt.at[0, w])
    cntv[pl.ds(0, 16)] = lax.broadcast_in_dim(g1, (16,), ())
    pltpu.sync_copy(cntv, bcnt.at[1, w])


_partition = pl.kernel(
    _partition_body,
    out_type=(
        jax.ShapeDtypeStruct((2, NW, REG), i32),
        jax.ShapeDtypeStruct((2, NW, REG), i32),
        jax.ShapeDtypeStruct((2, NW, REG), f32),
        jax.ShapeDtypeStruct((2, NW, 16), i32),
    ),
    mesh=_mesh,
    compiler_params=_params,
    scratch_types=[
        pltpu.VMEM((C,), i32),
        pltpu.VMEM((C,), i32),
        pltpu.VMEM((C,), f32),
        pltpu.VMEM((C,), i32),
        pltpu.VMEM((C,), i32),
        pltpu.VMEM((C,), f32),
        pltpu.VMEM((C,), i32),
        pltpu.VMEM((C,), i32),
        pltpu.VMEM((C,), f32),
        pltpu.VMEM((16,), i32),
    ],
)


def _layer_body(h0, h1, bcol, brow, bval, bcnt, o0, o1,
                cv0, cv1, dv0, dv1, vv0, vv1, rv0, rv1, cntv, acc,
                sg0, sg1, ss0, ss1):
    c = lax.axis_index("c")
    s = lax.axis_index("s")
    cv = [cv0, cv1]
    dv = [dv0, dv1]
    vv = [vv0, vv1]
    rv = [rv0, rv1]
    sg = [sg0, sg1]
    ss = [ss0, ss1]
    lane = lax.iota(i32, 16)

    def _stage(bi, p, w, ch):
        off = pl.multiple_of(ch * C, C)
        pltpu.sync_copy(bcol.at[p, w, pl.ds(off, C)], cv[bi])
        pltpu.sync_copy(brow.at[p, w, pl.ds(off, C)], dv[bi])
        pltpu.sync_copy(bval.at[p, w, pl.ds(off, C)], vv[bi])

    def _fire_gather(bi):
        @pl.when(c == 0)
        def _g0():
            @pl.loop(0, NSTREAM)
            def _g0j(j):
                pltpu.async_copy(h0.at[cv[bi].at[pl.ds(j * SUB, SUB)]],
                                 rv[bi].at[pl.ds(j * SUB, SUB)], sg[bi])

        @pl.when(c == 1)
        def _g1():
            @pl.loop(0, NSTREAM)
            def _g1j(j):
                pltpu.async_copy(h1.at[cv[bi].at[pl.ds(j * SUB, SUB)]],
                                 rv[bi].at[pl.ds(j * SUB, SUB)], sg[bi])

    def _wait_gather(bi):
        @pl.loop(0, NSTREAM)
        def _wgj(j):
            pltpu.make_async_copy(h0.at[cv[bi].at[pl.ds(j * SUB, SUB)]],
                                  rv[bi].at[pl.ds(j * SUB, SUB)],
                                  sg[bi]).wait()

    def _fire_scatter(bi):
        @pl.loop(0, NSTREAM)
        def _fsj(j):
            pltpu.async_copy(rv[bi].at[pl.ds(j * SUB, SUB)],
                             acc.at[dv[bi].at[pl.ds(j * SUB, SUB)]],
                             ss[bi], add=True)

    def _wait_scatter(bi):
        @pl.loop(0, NSTREAM)
        def _wsj(j):
            pltpu.make_async_copy(rv[bi].at[pl.ds(j * SUB, SUB)],
                                  acc.at[dv[bi].at[pl.ds(j * SUB, SUB)]],
                                  ss[bi]).wait()

    def _scale(bi):
        @pl.loop(0, C, step=16)
        def _sc(jj):
            v = vv[bi][pl.ds(jj, 16)]
            eidx16 = jj + lane
            for k2 in range(H):
                kf = jnp.full((16,), k2, i32)
                g = plsc.load_gather(rv[bi], [eidx16, kf])
                plsc.store_scatter(rv[bi], [eidx16, kf], g * v)

    @pl.loop(0, 2)
    def _pass(p):
        @pl.loop(0, C)
        def _zf(j):
            rv1[j, :] = jnp.zeros((16,), f32)

        zb = s * ZPT
        pltpu.sync_copy(rv1, acc.at[pl.ds(zb, C)])

        @pl.when(s < NS - 1)
        def _zr1():
            pltpu.sync_copy(rv1.at[pl.ds(0, ZPT - C)],
                            acc.at[pl.ds(zb + C, ZPT - C)])

        @pl.when(s == NS - 1)
        def _zr2():
            pltpu.sync_copy(rv1.at[pl.ds(0, ZPT_LAST - C)],
                            acc.at[pl.ds(zb + C, ZPT_LAST - C)])

        plsc.subcore_barrier()

        for rg in range(2):
            w = s + rg * NS
            pltpu.sync_copy(bcnt.at[p, w], cntv)
            cnt = jnp.max(cntv[pl.ds(0, 16)])
            nch = jnp.maximum((cnt + (C - 1)) >> 11, 1)

            _stage(0, p, w, 0)
            _fire_gather(0)

            @pl.loop(0, nch, step=2)
            def _chunk2(ch0):
                for b in range(2):
                    ch = ch0 + b
                    nb = 1 - b

                    @pl.when(ch < nch)
                    def _proc():
                        @pl.when(ch + 1 < nch)
                        def _fire_next():
                            @pl.when(ch >= 1)
                            def _ws():
                                _wait_scatter(nb)

                            _stage(nb, p, w, ch + 1)
                            _fire_gather(nb)

                        _wait_gather(b)
                        _scale(b)
                        _fire_scatter(b)

            _wait_scatter(0)

            @pl.when(nch >= 2)
            def _ws1():
                _wait_scatter(1)

        plsc.subcore_barrier()

        def _writeback(dst):
            base = s * ZPT
            pltpu.sync_copy(acc.at[pl.ds(base, C)],
                            dst.at[pl.ds(p * NH + base, C)])

            @pl.when(s < NS - 1)
            def _r1():
                pltpu.sync_copy(acc.at[pl.ds(base + C, ZPT - C)],
                                dst.at[pl.ds(p * NH + base + C, ZPT - C)])

            @pl.when(s == NS - 1)
            def _r2():
                pltpu.sync_copy(acc.at[pl.ds(base + C, ZPT_LAST - C)],
                                dst.at[pl.ds(p * NH + base + C, ZPT_LAST - C)])

        @pl.when(c == 0)
        def _w0():
            _writeback(o0)

        @pl.when(c == 1)
        def _w1():
            _writeback(o1)


_layer = pl.kernel(
    _layer_body,
    out_type=(
        jax.ShapeDtypeStruct((NN, H), f32),
        jax.ShapeDtypeStruct((NN, H), f32),
    ),
    mesh=_mesh,
    compiler_params=_params,
    scratch_types=[
        pltpu.VMEM((C,), i32),
        pltpu.VMEM((C,), i32),
        pltpu.VMEM((C,), i32),
        pltpu.VMEM((C,), i32),
        pltpu.VMEM((C,), f32),
        pltpu.VMEM((C,), f32),
        pltpu.VMEM((C, H), f32),
        pltpu.VMEM((C, H), f32),
        pltpu.VMEM((16,), i32),
        pltpu.VMEM_SHARED((ACC_ROWS, H), f32),
        pltpu.SemaphoreType.DMA,
        pltpu.SemaphoreType.DMA,
        pltpu.SemaphoreType.DMA,
        pltpu.SemaphoreType.DMA,
    ],
)

BPW = B // NW


def _score_body(e00, e01, e10, e11, e20, e21, e30, e31, uids, iids, out,
                idx_u, idx_i, gbuf, sc_v, sem):
    c = lax.axis_index("c")
    s = lax.axis_index("s")
    w = s * NC + c

    pltpu.sync_copy(uids.at[pl.ds(w * BPW, BPW)], idx_u)
    pltpu.sync_copy(iids.at[pl.ds(w * BPW, BPW)], idx_i)

    tabs = [e00, e01, e10, e11, e20, e21, e30, e31]
    descs = []
    for t in range(8):
        descs.append(pltpu.async_copy(tabs[t].at[idx_u], gbuf.at[t], sem))
        descs.append(pltpu.async_copy(tabs[t].at[idx_i], gbuf.at[8 + t], sem))
    for d in descs:
        d.wait()

    @pl.loop(0, BPW, step=16)
    def _dot(j0):
        lane = lax.iota(i32, 16)
        tot = jnp.zeros((16,), f32)
        for jj in range(16):
            j = j0 + jj
            u0 = gbuf[0, j, :] + gbuf[2, j, :] + gbuf[4, j, :] + gbuf[6, j, :]
            u1 = gbuf[1, j, :] + gbuf[3, j, :] + gbuf[5, j, :] + gbuf[7, j, :]
            i0 = gbuf[8, j, :] + gbuf[10, j, :] + gbuf[12, j, :] + gbuf[14, j, :]
            i1 = gbuf[9, j, :] + gbuf[11, j, :] + gbuf[13, j, :] + gbuf[15, j, :]
            p = u0 * i0 + u1 * i1
            ssum = jnp.sum(p) * (1.0 / 16.0)
            tot = jnp.where(lane == jj, lax.broadcast_in_dim(ssum, (16,), ()), tot)
        sc_v[pl.ds(j0, 16)] = tot

    pltpu.sync_copy(sc_v, out.at[pl.ds(w * BPW, BPW)])


_score = pl.kernel(
    _score_body,
    out_type=jax.ShapeDtypeStruct((B,), f32),
    mesh=_mesh,
    compiler_params=_params,
    scratch_types=[
        pltpu.VMEM((BPW,), i32),
        pltpu.VMEM((BPW,), i32),
        pltpu.VMEM((16, BPW, H), f32),
        pltpu.VMEM((BPW,), f32),
        pltpu.SemaphoreType.DMA,
    ],
)


def kernel(user_ids, item_ids, adj_indices, adj_values, user_emb_w, item_emb_w):
    row = adj_indices[0].astype(i32)
    col = adj_indices[1].astype(i32)
    pad = EPAD - E
    colp = jnp.concatenate([col, jnp.zeros((pad,), i32)])
    rowp = jnp.concatenate([row, jnp.zeros((pad,), i32)])
    valp = jnp.concatenate([adj_values.astype(f32), jnp.zeros((pad,), f32)])
    pe_idx = jnp.stack([colp.reshape(NW, PCH, C),
                        rowp.reshape(NW, PCH, C)], axis=2)
    pe_val = valp.reshape(NW, PCH, C)

    bcol, brow, bval, bcnt = _partition(pe_idx, pe_val)

    a0 = jnp.concatenate([user_emb_w[:, :H], item_emb_w[:, :H]], axis=0)
    a1 = jnp.concatenate([user_emb_w[:, H:], item_emb_w[:, H:]], axis=0)
    embs = [(a0, a1)]
    for _ in range(NLAYERS):
        a0, a1 = _layer(a0, a1, bcol, brow, bval, bcnt)
        embs.append((a0, a1))

    uids = user_ids.astype(i32)
    iids = item_ids.astype(i32) + NU
    return _score(embs[0][0], embs[0][1], embs[1][0], embs[1][1],
                  embs[2][0], embs[2][1], embs[3][0], embs[3][1],
                  uids, iids)


# speedup vs baseline: 1.5271x; 1.1976x over previous
import jax
import jax.numpy as jnp
from jax import lax
from jax.experimental import pallas as pl
from jax.experimental.pallas import tpu as pltpu
from jax.experimental.pallas import tpu_sc as plsc

f32 = jnp.float32
i32 = jnp.int32

NU = 50000
NI = 50000
NN = NU + NI
H = 16
NLAYERS = 3
E = 1600000
B = 4096

NC = 2
NS = 16
NW = NC * NS

SUB = 128
NSTREAM = 16
C = SUB * NSTREAM
PCH = 25
EPAD = NW * PCH * C
REG = 57344

LC = 1024
LSTREAM = LC // SUB
NH = NN // 2
ACC_ROWS = NH
ZPT = 3128
ZPT_LAST = NH - (NS - 1) * ZPT

_mesh = plsc.VectorSubcoreMesh(
    core_axis_name="c", subcore_axis_name="s", num_cores=NC, num_subcores=NS
)
_params = pltpu.CompilerParams(use_tc_tiling_on_sc=False,
                               needs_layout_passes=False)


def _partition_body(pe_idx, pe_val, bcol, brow, bval, bcnt,
                    icol, irow, ival, oc0, or0, ov0, oc1, or1, ov1, cntv):
    c = lax.axis_index("c")
    s = lax.axis_index("s")
    w = s * NC + c
    lane = lax.iota(i32, 16)
    zi = jnp.zeros((16,), i32)
    zf = jnp.zeros((16,), f32)

    @pl.loop(0, PCH, init_carry=(jnp.array(0, i32), jnp.array(0, i32)))
    def _chunks(ch, g):
        g0, g1 = g
        pltpu.sync_copy(pe_idx.at[w, ch, 0], icol)
        pltpu.sync_copy(pe_idx.at[w, ch, 1], irow)
        pltpu.sync_copy(pe_val.at[w, ch], ival)

        @pl.loop(0, C, step=16,
                 init_carry=(jnp.array(0, i32), jnp.array(0, i32)))
        def _groups(j0, pp):
            p0, p1 = pp
            cvec = icol[pl.ds(j0, 16)]
            rvec = irow[pl.ds(j0, 16)]
            vvec = ival[pl.ds(j0, 16)]
            m = rvec < NH
            cum = plsc.cumsum(jnp.where(m, 1, 0))
            pos0 = cum + (p0 - 1)
            plsc.store_scatter(oc0, [pos0], cvec, mask=m)
            plsc.store_scatter(or0, [pos0], rvec, mask=m)
            plsc.store_scatter(ov0, [pos0], vvec, mask=m)
            m1 = jnp.logical_not(m)
            pos1 = (lane + 1 - cum) + (p1 - 1)
            plsc.store_scatter(oc1, [pos1], cvec, mask=m1)
            plsc.store_scatter(or1, [pos1], rvec - NH, mask=m1)
            plsc.store_scatter(ov1, [pos1], vvec, mask=m1)
            n0 = jnp.max(cum)
            return (p0 + n0, p1 + (16 - n0))

        p0, p1 = _groups
        g0 = pl.multiple_of(g0, 128)
        g1 = pl.multiple_of(g1, 128)
        pp0 = (p0 + 127) & (-128)
        pp1 = (p1 + 127) & (-128)
        for g8 in range(8):
            posv = p0 + g8 * 16 + lane
            mk = posv < pp0
            plsc.store_scatter(oc0, [posv], zi, mask=mk)
            plsc.store_scatter(or0, [posv], zi, mask=mk)
            plsc.store_scatter(ov0, [posv], zf, mask=mk)
            posw = p1 + g8 * 16 + lane
            mk1 = posw < pp1
            plsc.store_scatter(oc1, [posw], zi, mask=mk1)
            plsc.store_scatter(or1, [posw], zi, mask=mk1)
            plsc.store_scatter(ov1, [posw], zf, mask=mk1)
        pltpu.sync_copy(oc0, bcol.at[0, w, pl.ds(g0, C)])
        pltpu.sync_copy(or0, brow.at[0, w, pl.ds(g0, C)])
        pltpu.sync_copy(ov0, bval.at[0, w, pl.ds(g0, C)])
        pltpu.sync_copy(oc1, bcol.at[1, w, pl.ds(g1, C)])
        pltpu.sync_copy(or1, brow.at[1, w, pl.ds(g1, C)])
        pltpu.sync_copy(ov1, bval.at[1, w, pl.ds(g1, C)])
        return (g0 + pp0, g1 + pp1)

    g0, g1 = _chunks
    g0 = pl.multiple_of(g0, 128)
    g1 = pl.multiple_of(g1, 128)

    @pl.loop(0, C, step=16)
    def _zz(j0):
        oc0[pl.ds(j0, 16)] = zi
        or0[pl.ds(j0, 16)] = zi
        ov0[pl.ds(j0, 16)] = zf

    pltpu.sync_copy(oc0, bcol.at[0, w, pl.ds(g0, C)])
    pltpu.sync_copy(or0, brow.at[0, w, pl.ds(g0, C)])
    pltpu.sync_copy(ov0, bval.at[0, w, pl.ds(g0, C)])
    pltpu.sync_copy(oc0, bcol.at[1, w, pl.ds(g1, C)])
    pltpu.sync_copy(or0, brow.at[1, w, pl.ds(g1, C)])
    pltpu.sync_copy(ov0, bval.at[1, w, pl.ds(g1, C)])

    cntv[pl.ds(0, 16)] = lax.broadcast_in_dim(g0, (16,), ())
    pltpu.sync_copy(cntv, bcnt.at[0, w])
    cntv[pl.ds(0, 16)] = lax.broadcast_in_dim(g1, (16,), ())
    pltpu.sync_copy(cntv, bcnt.at[1, w])


_partition = pl.kernel(
    _partition_body,
    out_type=(
        jax.ShapeDtypeStruct((2, NW, REG), i32),
        jax.ShapeDtypeStruct((2, NW, REG), i32),
        jax.ShapeDtypeStruct((2, NW, REG), f32),
        jax.ShapeDtypeStruct((2, NW, 16), i32),
    ),
    mesh=_mesh,
    compiler_params=_params,
    scratch_types=[
        pltpu.VMEM((C,), i32),
        pltpu.VMEM((C,), i32),
        pltpu.VMEM((C,), f32),
        pltpu.VMEM((C,), i32),
        pltpu.VMEM((C,), i32),
        pltpu.VMEM((C,), f32),
        pltpu.VMEM((C,), i32),
        pltpu.VMEM((C,), i32),
        pltpu.VMEM((C,), f32),
        pltpu.VMEM((16,), i32),
    ],
)


def _layer_body(h0, h1, bcol, brow, bval, bcnt, o0, o1,
                cv0, cv1, cv2, dv0, dv1, dv2, vv0, vv1, vv2,
                rv0, rv1, rv2, cntv, acc,
                sg0, sg1, sg2, ss0, ss1, ss2):
    c = lax.axis_index("c")
    s = lax.axis_index("s")
    cv = [cv0, cv1, cv2]
    dv = [dv0, dv1, dv2]
    vv = [vv0, vv1, vv2]
    rv = [rv0, rv1, rv2]
    sg = [sg0, sg1, sg2]
    ss = [ss0, ss1, ss2]
    lane = lax.iota(i32, 16)

    def _stage(bi, p, w, ch):
        off = pl.multiple_of(ch * LC, LC)
        pltpu.sync_copy(bcol.at[p, w, pl.ds(off, LC)], cv[bi])
        pltpu.sync_copy(brow.at[p, w, pl.ds(off, LC)], dv[bi])
        pltpu.sync_copy(bval.at[p, w, pl.ds(off, LC)], vv[bi])

    def _fire_gather(bi):
        @pl.when(c == 0)
        def _g0():
            @pl.loop(0, LSTREAM)
            def _g0j(j):
                pltpu.async_copy(h0.at[cv[bi].at[pl.ds(j * SUB, SUB)]],
                                 rv[bi].at[pl.ds(j * SUB, SUB)], sg[bi])

        @pl.when(c == 1)
        def _g1():
            @pl.loop(0, LSTREAM)
            def _g1j(j):
                pltpu.async_copy(h1.at[cv[bi].at[pl.ds(j * SUB, SUB)]],
                                 rv[bi].at[pl.ds(j * SUB, SUB)], sg[bi])

    def _wait_gather(bi):
        @pl.loop(0, LSTREAM)
        def _wgj(j):
            pltpu.make_async_copy(h0.at[cv[bi].at[pl.ds(j * SUB, SUB)]],
                                  rv[bi].at[pl.ds(j * SUB, SUB)],
                                  sg[bi]).wait()

    def _fire_scatter(bi):
        @pl.loop(0, LSTREAM)
        def _fsj(j):
            pltpu.async_copy(rv[bi].at[pl.ds(j * SUB, SUB)],
                             acc.at[dv[bi].at[pl.ds(j * SUB, SUB)]],
                             ss[bi], add=True)

    def _wait_scatter(bi):
        @pl.loop(0, LSTREAM)
        def _wsj(j):
            pltpu.make_async_copy(rv[bi].at[pl.ds(j * SUB, SUB)],
                                  acc.at[dv[bi].at[pl.ds(j * SUB, SUB)]],
                                  ss[bi]).wait()

    def _scale(bi):
        @pl.loop(0, LC, step=16)
        def _sc(jj):
            v = vv[bi][pl.ds(jj, 16)]
            eidx16 = jj + lane
            for k2 in range(H):
                kf = jnp.full((16,), k2, i32)
                g = plsc.load_gather(rv[bi], [eidx16, kf])
                plsc.store_scatter(rv[bi], [eidx16, kf], g * v)

    @pl.loop(0, 2)
    def _pass(p):
        @pl.loop(0, LC)
        def _zf(j):
            rv1[j, :] = jnp.zeros((16,), f32)

        zb = s * ZPT
        for m in range(3):
            pltpu.sync_copy(rv1, acc.at[pl.ds(zb + m * LC, LC)])

        @pl.when(s < NS - 1)
        def _zr1():
            pltpu.sync_copy(rv1.at[pl.ds(0, ZPT - 3 * LC)],
                            acc.at[pl.ds(zb + 3 * LC, ZPT - 3 * LC)])

        @pl.when(s == NS - 1)
        def _zr2():
            pltpu.sync_copy(rv1.at[pl.ds(0, ZPT_LAST - 3 * LC)],
                            acc.at[pl.ds(zb + 3 * LC, ZPT_LAST - 3 * LC)])

        plsc.subcore_barrier()

        for rg in range(2):
            w = s + rg * NS
            pltpu.sync_copy(bcnt.at[p, w], cntv)
            cnt = jnp.max(cntv[pl.ds(0, 16)])
            nch = jnp.maximum((cnt + (LC - 1)) >> 10, 1)

            _stage(0, p, w, 0)
            _fire_gather(0)

            @pl.when(nch >= 2)
            def _pro1():
                _stage(1, p, w, 1)
                _fire_gather(1)

            @pl.loop(0, nch, step=3)
            def _chunk3(ch0):
                for b in range(3):
                    ch = ch0 + b
                    nb = (b + 2) % 3

                    @pl.when(ch < nch)
                    def _proc():
                        @pl.when(ch + 2 < nch)
                        def _fire_next():
                            @pl.when(ch >= 1)
                            def _ws():
                                _wait_scatter(nb)

                            _stage(nb, p, w, ch + 2)
                            _fire_gather(nb)

                        _wait_gather(b)
                        _scale(b)
                        _fire_scatter(b)

            _wait_scatter(0)

            @pl.when(nch >= 2)
            def _ws1():
                _wait_scatter(1)

            @pl.when(nch >= 3)
            def _ws2():
                _wait_scatter(2)

        plsc.subcore_barrier()

        def _writeback(dst):
            base = s * ZPT
            pltpu.sync_copy(acc.at[pl.ds(base, 2048)],
                            dst.at[pl.ds(p * NH + base, 2048)])

            @pl.when(s < NS - 1)
            def _r1():
                pltpu.sync_copy(acc.at[pl.ds(base + 2048, ZPT - 2048)],
                                dst.at[pl.ds(p * NH + base + 2048, ZPT - 2048)])

            @pl.when(s == NS - 1)
            def _r2():
                pltpu.sync_copy(acc.at[pl.ds(base + 2048, ZPT_LAST - 2048)],
                                dst.at[pl.ds(p * NH + base + 2048, ZPT_LAST - 2048)])

        @pl.when(c == 0)
        def _w0():
            _writeback(o0)

        @pl.when(c == 1)
        def _w1():
            _writeback(o1)


_layer = pl.kernel(
    _layer_body,
    out_type=(
        jax.ShapeDtypeStruct((NN, H), f32),
        jax.ShapeDtypeStruct((NN, H), f32),
    ),
    mesh=_mesh,
    compiler_params=_params,
    scratch_types=[
        pltpu.VMEM((LC,), i32),
        pltpu.VMEM((LC,), i32),
        pltpu.VMEM((LC,), i32),
        pltpu.VMEM((LC,), i32),
        pltpu.VMEM((LC,), i32),
        pltpu.VMEM((LC,), i32),
        pltpu.VMEM((LC,), f32),
        pltpu.VMEM((LC,), f32),
        pltpu.VMEM((LC,), f32),
        pltpu.VMEM((LC, H), f32),
        pltpu.VMEM((LC, H), f32),
        pltpu.VMEM((LC, H), f32),
        pltpu.VMEM((16,), i32),
        pltpu.VMEM_SHARED((ACC_ROWS, H), f32),
        pltpu.SemaphoreType.DMA,
        pltpu.SemaphoreType.DMA,
        pltpu.SemaphoreType.DMA,
        pltpu.SemaphoreType.DMA,
        pltpu.SemaphoreType.DMA,
        pltpu.SemaphoreType.DMA,
    ],
)

BPW = B // NW


def _score_body(e00, e01, e10, e11, e20, e21, e30, e31, uids, iids, out,
                idx_u, idx_i, gbuf, sc_v, sem):
    c = lax.axis_index("c")
    s = lax.axis_index("s")
    w = s * NC + c

    pltpu.sync_copy(uids.at[pl.ds(w * BPW, BPW)], idx_u)
    pltpu.sync_copy(iids.at[pl.ds(w * BPW, BPW)], idx_i)

    tabs = [e00, e01, e10, e11, e20, e21, e30, e31]
    descs = []
    for t in range(8):
        descs.append(pltpu.async_copy(tabs[t].at[idx_u], gbuf.at[t], sem))
        descs.append(pltpu.async_copy(tabs[t].at[idx_i], gbuf.at[8 + t], sem))
    for d in descs:
        d.wait()

    @pl.loop(0, BPW, step=16)
    def _dot(j0):
        lane = lax.iota(i32, 16)
        tot = jnp.zeros((16,), f32)
        for jj in range(16):
            j = j0 + jj
            u0 = gbuf[0, j, :] + gbuf[2, j, :] + gbuf[4, j, :] + gbuf[6, j, :]
            u1 = gbuf[1, j, :] + gbuf[3, j, :] + gbuf[5, j, :] + gbuf[7, j, :]
            i0 = gbuf[8, j, :] + gbuf[10, j, :] + gbuf[12, j, :] + gbuf[14, j, :]
            i1 = gbuf[9, j, :] + gbuf[11, j, :] + gbuf[13, j, :] + gbuf[15, j, :]
            p = u0 * i0 + u1 * i1
            ssum = jnp.sum(p) * (1.0 / 16.0)
            tot = jnp.where(lane == jj, lax.broadcast_in_dim(ssum, (16,), ()), tot)
        sc_v[pl.ds(j0, 16)] = tot

    pltpu.sync_copy(sc_v, out.at[pl.ds(w * BPW, BPW)])


_score = pl.kernel(
    _score_body,
    out_type=jax.ShapeDtypeStruct((B,), f32),
    mesh=_mesh,
    compiler_params=_params,
    scratch_types=[
        pltpu.VMEM((BPW,), i32),
        pltpu.VMEM((BPW,), i32),
        pltpu.VMEM((16, BPW, H), f32),
        pltpu.VMEM((BPW,), f32),
        pltpu.SemaphoreType.DMA,
    ],
)


def kernel(user_ids, item_ids, adj_indices, adj_values, user_emb_w, item_emb_w):
    row = adj_indices[0].astype(i32)
    col = adj_indices[1].astype(i32)
    pad = EPAD - E
    colp = jnp.concatenate([col, jnp.zeros((pad,), i32)])
    rowp = jnp.concatenate([row, jnp.zeros((pad,), i32)])
    valp = jnp.concatenate([adj_values.astype(f32), jnp.zeros((pad,), f32)])
    pe_idx = jnp.stack([colp.reshape(NW, PCH, C),
                        rowp.reshape(NW, PCH, C)], axis=2)
    pe_val = valp.reshape(NW, PCH, C)

    bcol, brow, bval, bcnt = _partition(pe_idx, pe_val)

    a0 = jnp.concatenate([user_emb_w[:, :H], item_emb_w[:, :H]], axis=0)
    a1 = jnp.concatenate([user_emb_w[:, H:], item_emb_w[:, H:]], axis=0)
    embs = [(a0, a1)]
    for _ in range(NLAYERS):
        a0, a1 = _layer(a0, a1, bcol, brow, bval, bcnt)
        embs.append((a0, a1))

    uids = user_ids.astype(i32)
    iids = item_ids.astype(i32) + NU
    return _score(embs[0][0], embs[0][1], embs[1][0], embs[1][1],
                  embs[2][0], embs[2][1], embs[3][0], embs[3][1],
                  uids, iids)
